# Initial kernel scaffold; baseline (speedup 1.0000x reference)
import jax
import jax.numpy as jnp
from jax.experimental import pallas as pl


def kernel(f_atoms, f_bonds, a2b, b2a, b2revb, edit_data, last_edit_mask, prev_atom_hiddens, edit_table, W_i, W_h, W_o, W_vv, W_vc, W_conf, b_conf, W_ve, W_ve_out, W_ea1, b_ea1, W_ea2, b_ea2, W_al1, b_al1, W_al2, b_al2):
    raise NotImplementedError("write your pallas kernel here")


# speedup vs baseline: 1.0003x; 1.0003x over previous
import jax
import jax.numpy as jnp
from jax.experimental import pallas as pl

DEPTH = 3


def kernel(f_atoms, f_bonds, a2b, b2a, b2revb, edit_data, last_edit_mask, prev_atom_hiddens, edit_table, W_i, W_h, W_o, W_vv, W_vc, W_conf, b_conf, W_ve, W_ve_out, W_ea1, b_ea1, W_ea2, b_ea2, W_al1, b_al1, W_al2, b_al2):
    h0 = jax.nn.relu(f_bonds @ W_i)
    h = h0
    for _ in range(DEPTH - 1):
        nei = h[a2b].sum(axis=1)
        msg = nei[b2a] - h[b2revb]
        h = jax.nn.relu(h0 + msg @ W_h)
    nei = h[a2b].sum(axis=1)
    a_feats = jax.nn.relu(jnp.concatenate([f_atoms, nei], axis=-1) @ W_o)
    edit_emb = edit_table[edit_data].at[0].set(0.0)
    mask_emb = edit_table[last_edit_mask].at[0].set(0.0)
    atom_feats = jax.nn.relu(prev_atom_hiddens @ W_vv + a_feats @ W_vc)
    a2a = b2a[a2b]
    a2a = jnp.concatenate([jnp.arange(a2b.shape[0])[:, None], a2a], axis=1)
    imp = jnp.concatenate([edit_emb, atom_feats], axis=-1) @ W_conf + b_conf
    imp_g = imp[a2a]
    aee = edit_emb[a2a]
    aee = aee.at[0].set(0.0)
    aee = (imp_g * aee).sum(axis=1)
    ve = jax.nn.relu(aee) @ W_ve
    ve_out = jax.nn.relu(mask_emb) @ W_ve_out
    cat = jnp.concatenate([atom_feats, ve, ve_out], axis=-1)
    h1 = jax.nn.relu(cat @ W_ea1 + b_ea1)
    af2 = jax.nn.relu(h1 @ W_ea2 + b_ea2)
    scores = jax.nn.relu(af2 @ W_al1 + b_al1) @ W_al2 + b_al2
    return scores


# speedup vs baseline: 1.4478x; 1.4473x over previous
import functools

import jax
import jax.numpy as jnp
from jax import lax
from jax.experimental import pallas as pl
from jax.experimental.pallas import tpu as pltpu
from jax.experimental.pallas import tpu_sc as plsc

N = 10000
E = 160000
MAXNB = 8
DEPTH = 3
H = 256
NW = 32

_MESH = plsc.VectorSubcoreMesh(core_axis_name="c", subcore_axis_name="s",
                               num_cores=2, num_subcores=16)


def _wid():
    return lax.axis_index("s") * 2 + lax.axis_index("c")


def _ceil_div(a, b):
    return (a + b - 1) // b


def sc_gathersum(table, idx_flat):
    CH = 16
    NBLK = N // CH
    ITERS = _ceil_div(NBLK, NW)

    def body(table_hbm, idx_hbm, out_hbm, idx_v, rows_v, acc_v, sem):
        w = _wid()

        def blk(i, _):
            b = w + i * NW

            @pl.when(b < NBLK)
            def _():
                pltpu.sync_copy(idx_hbm.at[pl.ds(b * CH * 8, CH * 8)], idx_v)
                pltpu.async_copy(table_hbm.at[idx_v], rows_v, sem).wait()

                def atom(a, _2):
                    base = a * 8
                    for t in range(H // 16):
                        sl = pl.ds(t * 16, 16)
                        v = rows_v[base, sl]
                        for j in range(1, 8):
                            v = v + rows_v[base + j, sl]
                        acc_v[a, sl] = v
                    return 0

                lax.fori_loop(0, CH, atom, 0)
                pltpu.sync_copy(acc_v, out_hbm.at[pl.ds(b * CH, CH)])
            return 0

        lax.fori_loop(0, ITERS, blk, 0)

    f = pl.kernel(
        body,
        out_type=jax.ShapeDtypeStruct((N, H), jnp.float32),
        mesh=_MESH,
        scratch_types=[
            pltpu.VMEM((CH * 8,), jnp.int32),
            pltpu.VMEM((CH * 8, H), jnp.float32),
            pltpu.VMEM((CH, H), jnp.float32),
            pltpu.SemaphoreType.DMA,
        ],
    )
    return f(table, idx_flat)


def sc_msg(nei, h, b2a, b2revb):
    CB = 128
    NBLK = E // CB
    ITERS = _ceil_div(NBLK, NW)

    def body(nei_hbm, h_hbm, b2a_hbm, brev_hbm, out_hbm,
             ia_v, ib_v, ra_v, rb_v, sema, semb):
        w = _wid()

        def blk(i, _):
            b = w + i * NW

            @pl.when(b < NBLK)
            def _():
                pltpu.sync_copy(b2a_hbm.at[pl.ds(b * CB, CB)], ia_v)
                pltpu.sync_copy(brev_hbm.at[pl.ds(b * CB, CB)], ib_v)
                cpa = pltpu.async_copy(nei_hbm.at[ia_v], ra_v, sema)
                cpb = pltpu.async_copy(h_hbm.at[ib_v], rb_v, semb)
                cpa.wait()
                cpb.wait()

                def bond(a, _2):
                    for t in range(H // 16):
                        sl = pl.ds(t * 16, 16)
                        ra_v[a, sl] = ra_v[a, sl] - rb_v[a, sl]
                    return 0

                lax.fori_loop(0, CB, bond, 0)
                pltpu.sync_copy(ra_v, out_hbm.at[pl.ds(b * CB, CB)])
            return 0

        lax.fori_loop(0, ITERS, blk, 0)

    f = pl.kernel(
        body,
        out_type=jax.ShapeDtypeStruct((E, H), jnp.float32),
        mesh=_MESH,
        scratch_types=[
            pltpu.VMEM((CB,), jnp.int32),
            pltpu.VMEM((CB,), jnp.int32),
            pltpu.VMEM((CB, H), jnp.float32),
            pltpu.VMEM((CB, H), jnp.float32),
            pltpu.SemaphoreType.DMA,
            pltpu.SemaphoreType.DMA,
        ],
    )
    return f(nei, h, b2a, b2revb)


def sc_wsum(weighted, a2b_flat, b2a):
    CH = 16
    NBLK = N // CH
    ITERS = _ceil_div(NBLK, NW)

    def body(w_hbm, a2b_hbm, b2a_hbm, out_hbm,
             idx_v, idx2_v, rows_v, self_v, acc_v, sem):
        w = _wid()

        def blk(i, _):
            b = w + i * NW

            @pl.when(b < NBLK)
            def _():
                pltpu.sync_copy(a2b_hbm.at[pl.ds(b * CH * 8, CH * 8)], idx_v)
                pltpu.async_copy(b2a_hbm.at[idx_v], idx2_v, sem).wait()
                cp = pltpu.async_copy(w_hbm.at[idx2_v], rows_v, sem)
                pltpu.sync_copy(w_hbm.at[pl.ds(b * CH, CH)], self_v)
                cp.wait()

                def atom(a, _2):
                    base = a * 8
                    for t in range(H // 16):
                        sl = pl.ds(t * 16, 16)
                        v = self_v[a, sl]
                        for j in range(8):
                            v = v + rows_v[base + j, sl]
                        acc_v[a, sl] = v
                    return 0

                lax.fori_loop(0, CH, atom, 0)

                @pl.when(b == 0)
                def _():
                    for t in range(H // 16):
                        acc_v[0, pl.ds(t * 16, 16)] = jnp.zeros((16,), jnp.float32)

                pltpu.sync_copy(acc_v, out_hbm.at[pl.ds(b * CH, CH)])
            return 0

        lax.fori_loop(0, ITERS, blk, 0)

    f = pl.kernel(
        body,
        out_type=jax.ShapeDtypeStruct((N, H), jnp.float32),
        mesh=_MESH,
        scratch_types=[
            pltpu.VMEM((CH * 8,), jnp.int32),
            pltpu.VMEM((CH * 8,), jnp.int32),
            pltpu.VMEM((CH * 8, H), jnp.float32),
            pltpu.VMEM((CH, H), jnp.float32),
            pltpu.VMEM((CH, H), jnp.float32),
            pltpu.SemaphoreType.DMA,
        ],
    )
    return f(weighted, a2b_flat, b2a)


def _full(shape):
    return pl.BlockSpec(shape, lambda i: (0, 0))


def tc_mm_relu(x, w, m_blk):
    M, K = x.shape
    Kw, Nw = w.shape

    def body(x_ref, w_ref, o_ref):
        o_ref[:] = jax.nn.relu(
            jnp.dot(x_ref[:], w_ref[:], preferred_element_type=jnp.float32))

    return pl.pallas_call(
        body,
        grid=(M // m_blk,),
        in_specs=[pl.BlockSpec((m_blk, K), lambda i: (i, 0)), _full((Kw, Nw))],
        out_specs=pl.BlockSpec((m_blk, Nw), lambda i: (i, 0)),
        out_shape=jax.ShapeDtypeStruct((M, Nw), jnp.float32),
    )(x, w)


def tc_mm_residual_relu(msg, h0, w, m_blk):
    M, K = msg.shape

    def body(m_ref, h0_ref, w_ref, o_ref):
        o_ref[:] = jax.nn.relu(
            h0_ref[:] + jnp.dot(m_ref[:], w_ref[:],
                                preferred_element_type=jnp.float32))

    return pl.pallas_call(
        body,
        grid=(M // m_blk,),
        in_specs=[pl.BlockSpec((m_blk, K), lambda i: (i, 0)),
                  pl.BlockSpec((m_blk, H), lambda i: (i, 0)),
                  _full((K, H))],
        out_specs=pl.BlockSpec((m_blk, H), lambda i: (i, 0)),
        out_shape=jax.ShapeDtypeStruct((M, H), jnp.float32),
    )(msg, h0, w)


def tc_atoms(f_atoms, nei, prev, eids, mids, table, W_o, W_vv, W_vc,
             W_conf, b_conf, W_ve_out):
    MB = 1000
    AF = f_atoms.shape[1]
    V = table.shape[0]

    def body(fa, ne, pv, ei, mi, tb, wo, wvv, wvc, wc, bc, wveo,
             atf_o, wgt_o, veo_o):
        pid = pl.program_id(0)
        wo_a = wo[:]
        af = jax.nn.relu(
            jnp.dot(fa[:], wo_a[:AF], preferred_element_type=jnp.float32)
            + jnp.dot(ne[:], wo_a[AF:], preferred_element_type=jnp.float32))
        atf = jax.nn.relu(
            jnp.dot(pv[:], wvv[:], preferred_element_type=jnp.float32)
            + jnp.dot(af, wvc[:], preferred_element_type=jnp.float32))

        glob = lax.broadcasted_iota(jnp.int32, (MB, H), 0) + pid * MB
        iota_v = lax.broadcasted_iota(jnp.int32, (MB, V), 1)
        oh_e = (ei[:] == iota_v).astype(jnp.float32)
        ee = jnp.dot(oh_e, tb[:], preferred_element_type=jnp.float32)
        ee = jnp.where(glob == 0, 0.0, ee)
        oh_m = (mi[:] == iota_v).astype(jnp.float32)
        me = jnp.dot(oh_m, tb[:], preferred_element_type=jnp.float32)
        me = jnp.where(glob == 0, 0.0, me)

        wc_a = wc[:]
        imp = (jnp.dot(ee, wc_a[:H], preferred_element_type=jnp.float32)
               + jnp.dot(atf, wc_a[H:], preferred_element_type=jnp.float32)
               + bc[0, 0])
        atf_o[:] = atf
        wgt_o[:] = imp * ee
        veo_o[:] = jnp.dot(jax.nn.relu(me), wveo[:],
                           preferred_element_type=jnp.float32)

    return pl.pallas_call(
        body,
        grid=(N // MB,),
        in_specs=[
            pl.BlockSpec((MB, AF), lambda i: (i, 0)),
            pl.BlockSpec((MB, H), lambda i: (i, 0)),
            pl.BlockSpec((MB, H), lambda i: (i, 0)),
            pl.BlockSpec((MB, 1), lambda i: (i, 0)),
            pl.BlockSpec((MB, 1), lambda i: (i, 0)),
            _full((V, H)),
            _full((AF + H, H)),
            _full((H, H)),
            _full((H, H)),
            _full((2 * H, 1)),
            _full((1, 1)),
            _full((H, H)),
        ],
        out_specs=[pl.BlockSpec((MB, H), lambda i: (i, 0))] * 3,
        out_shape=[jax.ShapeDtypeStruct((N, H), jnp.float32)] * 3,
    )(f_atoms, nei, prev, eids, mids, table, W_o, W_vv, W_vc, W_conf,
      b_conf, W_ve_out)


def tc_head(atf, aee, veo, W_ve, W_ea1, b_ea1, W_ea2, b_ea2,
            W_al1, b_al1, W_al2, b_al2):
    MB = 1000
    MLP = W_ea1.shape[1]
    AOUT = W_al2.shape[1]

    def body(at, ae, vo, wve, wea1, bea1, wea2, bea2, wal1, bal1,
             wal2, bal2, o_ref):
        ve = jnp.dot(jax.nn.relu(ae[:]), wve[:],
                     preferred_element_type=jnp.float32)
        wea1_a = wea1[:]
        h1 = jax.nn.relu(
            jnp.dot(at[:], wea1_a[:H], preferred_element_type=jnp.float32)
            + jnp.dot(ve, wea1_a[H:2 * H], preferred_element_type=jnp.float32)
            + jnp.dot(vo[:], wea1_a[2 * H:], preferred_element_type=jnp.float32)
            + bea1[:])
        af2 = jax.nn.relu(
            jnp.dot(h1, wea2[:], preferred_element_type=jnp.float32) + bea2[:])
        s1 = jax.nn.relu(
            jnp.dot(af2, wal1[:], preferred_element_type=jnp.float32) + bal1[:])
        o_ref[:] = (jnp.dot(s1, wal2[:], preferred_element_type=jnp.float32)
                    + bal2[:])

    return pl.pallas_call(
        body,
        grid=(N // MB,),
        in_specs=[
            pl.BlockSpec((MB, H), lambda i: (i, 0)),
            pl.BlockSpec((MB, H), lambda i: (i, 0)),
            pl.BlockSpec((MB, H), lambda i: (i, 0)),
            _full((H, H)),
            _full((3 * H, MLP)),
            _full((1, MLP)),
            _full((MLP, H)),
            _full((1, H)),
            _full((H, MLP)),
            _full((1, MLP)),
            _full((MLP, AOUT)),
            _full((1, AOUT)),
        ],
        out_specs=pl.BlockSpec((MB, AOUT), lambda i: (i, 0)),
        out_shape=jax.ShapeDtypeStruct((N, AOUT), jnp.float32),
    )(atf, aee, veo, W_ve, W_ea1, b_ea1, W_ea2, b_ea2, W_al1, b_al1,
      W_al2, b_al2)


def kernel(f_atoms, f_bonds, a2b, b2a, b2revb, edit_data, last_edit_mask,
           prev_atom_hiddens, edit_table, W_i, W_h, W_o, W_vv, W_vc, W_conf,
           b_conf, W_ve, W_ve_out, W_ea1, b_ea1, W_ea2, b_ea2, W_al1, b_al1,
           W_al2, b_al2):
    a2b_flat = a2b.reshape(-1).astype(jnp.int32)
    b2a32 = b2a.astype(jnp.int32)
    b2revb32 = b2revb.astype(jnp.int32)

    h0 = tc_mm_relu(f_bonds, W_i, 640)
    h = h0
    for _ in range(DEPTH - 1):
        nei = sc_gathersum(h, a2b_flat)
        msg = sc_msg(nei, h, b2a32, b2revb32)
        h = tc_mm_residual_relu(msg, h0, W_h, 640)
    nei = sc_gathersum(h, a2b_flat)

    atf, weighted, veo = tc_atoms(
        f_atoms, nei, prev_atom_hiddens,
        edit_data.reshape(-1, 1).astype(jnp.int32),
        last_edit_mask.reshape(-1, 1).astype(jnp.int32),
        edit_table, W_o, W_vv, W_vc, W_conf, b_conf.reshape(1, 1), W_ve_out)

    aee = sc_wsum(weighted, a2b_flat, b2a32)

    return tc_head(atf, aee, veo, W_ve, W_ea1, b_ea1.reshape(1, -1),
                   W_ea2, b_ea2.reshape(1, -1), W_al1, b_al1.reshape(1, -1),
                   W_al2, b_al2.reshape(1, -1))
